# Initial kernel scaffold; baseline (speedup 1.0000x reference)
import jax
import jax.numpy as jnp
from jax.experimental import pallas as pl


def kernel(x, edge_index, edge_attr, ln_scale, ln_bias, W1, b1, W2, b2):
    raise NotImplementedError("write your pallas kernel here")


# speedup vs baseline: 2.5097x; 2.5097x over previous
import functools

import jax
import jax.numpy as jnp
from jax import lax
from jax.experimental import pallas as pl
from jax.experimental.pallas import tpu as pltpu
from jax.experimental.pallas import tpu_sc as plsc

_N = 10000
_NP = 10240
_E = 160000
_D = 256

_NC = 2
_NS = 16
_DH = _D // 2
_CH = 80
_EPT = _E // _NS
_NCH = _EPT // _CH
_RPT = _NP // _NS
_ZR = 128
_VL = 16


@functools.partial(
    pl.kernel,
    out_type=(
        jax.ShapeDtypeStruct((_NC, _NP, _DH), jnp.float32),
        jax.ShapeDtypeStruct((_NP,), jnp.float32),
    ),
    mesh=plsc.VectorSubcoreMesh(core_axis_name="c", subcore_axis_name="s"),
    scratch_types=[
        pltpu.VMEM((_CH,), jnp.int32),
        pltpu.VMEM((_CH,), jnp.int32),
        pltpu.VMEM((_CH, _DH), jnp.float32),
        pltpu.VMEM((_CH, _DH), jnp.float32),
        pltpu.VMEM((_CH,), jnp.float32),
        pltpu.VMEM((_ZR, _DH), jnp.float32),
        pltpu.VMEM((_RPT,), jnp.float32),
        pltpu.VMEM_SHARED((_NP, _DH), jnp.float32),
        pltpu.VMEM_SHARED((_NP,), jnp.float32),
        pltpu.SemaphoreType.DMA,
    ],
)
def _sc_aggregate(xs_hbm, src_hbm, dst_hbm, ea_hbm,
                  msg_hbm, inv_hbm,
                  src_v, dst_v, xj_v, ea_v, ones_v, zb_v, cnt_v,
                  acc_s, cnt_s, sem):
    c = lax.axis_index("c")
    s = lax.axis_index("s")

    zero16 = jnp.zeros((_VL,), jnp.float32)
    one16 = jnp.full((_VL,), 1.0, jnp.float32)

    def fill_zb(i, _):
        r = i // (_DH // _VL)
        o = (i % (_DH // _VL)) * _VL
        zb_v[r, pl.ds(o, _VL)] = zero16
        return 0
    lax.fori_loop(0, _ZR * (_DH // _VL), fill_zb, 0)

    def fill_ones(i, _):
        ones_v[pl.ds(i * _VL, _VL)] = one16
        return 0
    lax.fori_loop(0, _CH // _VL, fill_ones, 0)

    row0 = s * _RPT
    for q in range(_RPT // _ZR):
        pltpu.sync_copy(zb_v, acc_s.at[pl.ds(row0 + q * _ZR, _ZR), :])
        pltpu.sync_copy(zb_v.at[q], cnt_s.at[pl.ds(row0 + q * _ZR, _ZR)])

    plsc.subcore_barrier()

    ebase = s * _EPT

    def chunk(k, _):
        eb = ebase + k * _CH
        pltpu.sync_copy(src_hbm.at[pl.ds(eb, _CH)], src_v)
        pltpu.sync_copy(dst_hbm.at[pl.ds(eb, _CH)], dst_v)
        gcp = pltpu.async_copy(xs_hbm.at[c].at[src_v], xj_v, sem)
        pltpu.sync_copy(ea_hbm.at[pl.ds(eb, _CH), pl.ds(c * _DH, _DH)], ea_v)
        gcp.wait()

        def crow(r, _):
            for j in range(_DH // _VL):
                o = j * _VL
                v = xj_v[r, pl.ds(o, _VL)] + ea_v[r, pl.ds(o, _VL)]
                xj_v[r, pl.ds(o, _VL)] = v / (1.0 + jnp.exp(-v))
            return 0
        lax.fori_loop(0, _CH, crow, 0)

        pltpu.sync_copy(xj_v, acc_s.at[dst_v], add=True)
        pltpu.sync_copy(ones_v, cnt_s.at[dst_v], add=True)
        return 0

    lax.fori_loop(0, _NCH, chunk, 0)

    plsc.subcore_barrier()

    pltpu.sync_copy(acc_s.at[pl.ds(row0, _RPT), :],
                    msg_hbm.at[c, pl.ds(row0, _RPT), :])

    @pl.when(c == 0)
    def _():
        pltpu.sync_copy(cnt_s.at[pl.ds(row0, _RPT)], cnt_v)

        def recip(i, _):
            o = i * _VL
            v = cnt_v[pl.ds(o, _VL)]
            cnt_v[pl.ds(o, _VL)] = 1.0 / jnp.maximum(v, 1.0)
            return 0
        lax.fori_loop(0, _RPT // _VL, recip, 0)

        pltpu.sync_copy(cnt_v, inv_hbm.at[pl.ds(row0, _RPT)])


def _tc_body(x_ref, m0_ref, m1_ref, inv_ref, w1_ref, b1_ref, w2_ref, b2_ref,
             o_ref):
    x = x_ref[...]
    aggr = jnp.concatenate([m0_ref[...], m1_ref[...]], axis=1) * inv_ref[...]
    z = x + aggr
    h = jnp.dot(z, w1_ref[...], preferred_element_type=jnp.float32) + b1_ref[...]
    h = h / (1.0 + jnp.exp(-h))
    h = jnp.dot(h, w2_ref[...], preferred_element_type=jnp.float32) + b2_ref[...]
    o_ref[...] = x + h


_TB = 1000


def _tc_update(x, m0, m1, inv, W1, b1, W2, b2):
    grid = (_N // _TB,)
    return pl.pallas_call(
        _tc_body,
        grid=grid,
        in_specs=[
            pl.BlockSpec((_TB, _D), lambda i: (i, 0)),
            pl.BlockSpec((_TB, _DH), lambda i: (i, 0)),
            pl.BlockSpec((_TB, _DH), lambda i: (i, 0)),
            pl.BlockSpec((_TB, 1), lambda i: (i, 0)),
            pl.BlockSpec((_D, _D), lambda i: (0, 0)),
            pl.BlockSpec((1, _D), lambda i: (0, 0)),
            pl.BlockSpec((_D, _D), lambda i: (0, 0)),
            pl.BlockSpec((1, _D), lambda i: (0, 0)),
        ],
        out_specs=pl.BlockSpec((_TB, _D), lambda i: (i, 0)),
        out_shape=jax.ShapeDtypeStruct((_N, _D), jnp.float32),
    )(x, m0, m1, inv, W1, b1, W2, b2)


def kernel(x, edge_index, edge_attr, ln_scale, ln_bias, W1, b1, W2, b2):
    xs = jnp.transpose(x.reshape(_N, _NC, _DH), (1, 0, 2))
    src = edge_index[0]
    dst = edge_index[1]
    msg, inv = _sc_aggregate(xs, src, dst, edge_attr)
    return _tc_update(x, msg[0], msg[1], inv[:_N, None],
                      W1, b1.reshape(1, _D), W2, b2.reshape(1, _D))


# speedup vs baseline: 3.7686x; 1.5016x over previous
import functools

import jax
import jax.numpy as jnp
from jax import lax
from jax.experimental import pallas as pl
from jax.experimental.pallas import tpu as pltpu
from jax.experimental.pallas import tpu_sc as plsc

_N = 10000
_NP = 10240
_E = 160000
_D = 256

_NC = 2
_NS = 16
_DH = _D // 2
_CH = 80
_EPT = _E // _NS
_NCH = _EPT // _CH
_RPT = _NP // _NS
_ZR = 16
_VL = 16


@functools.partial(
    pl.kernel,
    out_type=(
        jax.ShapeDtypeStruct((_NC, _NP, _DH), jnp.float32),
        jax.ShapeDtypeStruct((_NP,), jnp.float32),
    ),
    mesh=plsc.VectorSubcoreMesh(core_axis_name="c", subcore_axis_name="s"),
    scratch_types=[
        pltpu.VMEM((_CH,), jnp.int32),
        pltpu.VMEM((_CH,), jnp.int32),
        pltpu.VMEM((_CH,), jnp.int32),
        pltpu.VMEM((_CH,), jnp.int32),
        pltpu.VMEM((_CH, _DH), jnp.float32),
        pltpu.VMEM((_CH, _DH), jnp.float32),
        pltpu.VMEM((_CH, _DH), jnp.float32),
        pltpu.VMEM((_CH, _DH), jnp.float32),
        pltpu.VMEM((_CH,), jnp.float32),
        pltpu.VMEM((_ZR, _DH), jnp.float32),
        pltpu.VMEM((_RPT,), jnp.float32),
        pltpu.VMEM_SHARED((_NP, _DH), jnp.float32),
        pltpu.VMEM_SHARED((_NP,), jnp.float32),
        pltpu.SemaphoreType.DMA,
        pltpu.SemaphoreType.DMA,
        pltpu.SemaphoreType.DMA,
        pltpu.SemaphoreType.DMA,
        pltpu.SemaphoreType.DMA,
        pltpu.SemaphoreType.DMA,
    ],
)
def _sc_aggregate(xs_hbm, src_hbm, dst_hbm, ea_hbm,
                  msg_hbm, inv_hbm,
                  src_v0, dst_v0, src_v1, dst_v1,
                  xj_v0, ea_v0, xj_v1, ea_v1,
                  ones_v, zb_v, cnt_v,
                  acc_s, cnt_s,
                  sem_i0, sem_i1, sem_g0, sem_g1, sem_e0, sem_e1):
    c = lax.axis_index("c")
    s = lax.axis_index("s")

    zero16 = jnp.zeros((_VL,), jnp.float32)
    one16 = jnp.full((_VL,), 1.0, jnp.float32)

    def fill_zb(i, _):
        r = i // (_DH // _VL)
        o = (i % (_DH // _VL)) * _VL
        zb_v[r, pl.ds(o, _VL)] = zero16
        return 0
    lax.fori_loop(0, _ZR * (_DH // _VL), fill_zb, 0)

    def fill_ones(i, _):
        ones_v[pl.ds(i * _VL, _VL)] = one16
        return 0
    lax.fori_loop(0, _CH // _VL, fill_ones, 0)

    row0 = s * _RPT
    for q in range(_RPT // _ZR):
        pltpu.sync_copy(zb_v, acc_s.at[pl.ds(row0 + q * _ZR, _ZR), :])
    for q in range(_RPT // _DH):
        pltpu.sync_copy(zb_v.at[0], cnt_s.at[pl.ds(row0 + q * _DH, _DH)])

    plsc.subcore_barrier()

    ebase = s * _EPT
    bufs = (
        (src_v0, dst_v0, xj_v0, ea_v0, sem_i0, sem_g0, sem_e0),
        (src_v1, dst_v1, xj_v1, ea_v1, sem_i1, sem_g1, sem_e1),
    )

    def issue_idx(k, b):
        eb = ebase + k * _CH
        pltpu.async_copy(src_hbm.at[pl.ds(eb, _CH)], b[0], b[4])
        pltpu.async_copy(dst_hbm.at[pl.ds(eb, _CH)], b[1], b[4])

    def wait_idx(b):
        pltpu.make_async_copy(src_hbm.at[pl.ds(0, _CH)], b[0], b[4]).wait()
        pltpu.make_async_copy(dst_hbm.at[pl.ds(0, _CH)], b[1], b[4]).wait()

    def issue_ge(k, b):
        eb = ebase + k * _CH
        pltpu.async_copy(xs_hbm.at[c].at[b[0]], b[2], b[5])
        pltpu.async_copy(ea_hbm.at[pl.ds(eb, _CH), pl.ds(c * _DH, _DH)],
                         b[3], b[6])

    def wait_ge(b):
        pltpu.make_async_copy(xs_hbm.at[c].at[b[0]], b[2], b[5]).wait()
        pltpu.make_async_copy(ea_hbm.at[pl.ds(0, _CH), pl.ds(0, _DH)],
                              b[3], b[6]).wait()

    def chunk_step(k, p):
        b = bufs[p]
        bn = bufs[1 - p]

        @pl.when(k + 1 < _NCH)
        def _():
            wait_idx(bn)
            issue_ge(k + 1, bn)

        wait_ge(b)

        xj_v, ea_v = b[2], b[3]

        def crow(r, _):
            for j in range(_DH // _VL):
                o = j * _VL
                v = xj_v[r, pl.ds(o, _VL)] + ea_v[r, pl.ds(o, _VL)]
                xj_v[r, pl.ds(o, _VL)] = v / (1.0 + jnp.exp(-v))
            return 0
        lax.fori_loop(0, _CH, crow, 0)

        pltpu.sync_copy(xj_v, acc_s.at[b[1]], add=True)
        pltpu.sync_copy(ones_v, cnt_s.at[b[1]], add=True)

        @pl.when(k + 2 < _NCH)
        def _():
            issue_idx(k + 2, b)

    pltpu.sync_copy(src_hbm.at[pl.ds(ebase, _CH)], src_v0)
    pltpu.sync_copy(dst_hbm.at[pl.ds(ebase, _CH)], dst_v0)
    issue_ge(0, bufs[0])
    issue_idx(1, bufs[1])

    def pair(j, _):
        chunk_step(2 * j, 0)
        chunk_step(2 * j + 1, 1)
        return 0
    lax.fori_loop(0, (_NCH - 1) // 2, pair, 0)
    chunk_step(_NCH - 1, 0)

    plsc.subcore_barrier()

    pltpu.sync_copy(acc_s.at[pl.ds(row0, _RPT), :],
                    msg_hbm.at[c, pl.ds(row0, _RPT), :])

    @pl.when(c == 0)
    def _():
        pltpu.sync_copy(cnt_s.at[pl.ds(row0, _RPT)], cnt_v)

        def recip(i, _):
            o = i * _VL
            v = cnt_v[pl.ds(o, _VL)]
            cnt_v[pl.ds(o, _VL)] = 1.0 / jnp.maximum(v, 1.0)
            return 0
        lax.fori_loop(0, _RPT // _VL, recip, 0)

        pltpu.sync_copy(cnt_v, inv_hbm.at[pl.ds(row0, _RPT)])


def _tc_body(x_ref, m0_ref, m1_ref, inv_ref, w1_ref, b1_ref, w2_ref, b2_ref,
             o_ref):
    x = x_ref[...]
    aggr = jnp.concatenate([m0_ref[...], m1_ref[...]], axis=1) * inv_ref[...]
    z = x + aggr
    h = jnp.dot(z, w1_ref[...], preferred_element_type=jnp.float32) + b1_ref[...]
    h = h / (1.0 + jnp.exp(-h))
    h = jnp.dot(h, w2_ref[...], preferred_element_type=jnp.float32) + b2_ref[...]
    o_ref[...] = x + h


_TB = 1000


def _tc_update(x, m0, m1, inv, W1, b1, W2, b2):
    grid = (_N // _TB,)
    return pl.pallas_call(
        _tc_body,
        grid=grid,
        in_specs=[
            pl.BlockSpec((_TB, _D), lambda i: (i, 0)),
            pl.BlockSpec((_TB, _DH), lambda i: (i, 0)),
            pl.BlockSpec((_TB, _DH), lambda i: (i, 0)),
            pl.BlockSpec((_TB, 1), lambda i: (i, 0)),
            pl.BlockSpec((_D, _D), lambda i: (0, 0)),
            pl.BlockSpec((1, _D), lambda i: (0, 0)),
            pl.BlockSpec((_D, _D), lambda i: (0, 0)),
            pl.BlockSpec((1, _D), lambda i: (0, 0)),
        ],
        out_specs=pl.BlockSpec((_TB, _D), lambda i: (i, 0)),
        out_shape=jax.ShapeDtypeStruct((_N, _D), jnp.float32),
    )(x, m0, m1, inv, W1, b1, W2, b2)


def kernel(x, edge_index, edge_attr, ln_scale, ln_bias, W1, b1, W2, b2):
    xs = jnp.transpose(x.reshape(_N, _NC, _DH), (1, 0, 2))
    src = edge_index[0]
    dst = edge_index[1]
    msg, inv = _sc_aggregate(xs, src, dst, edge_attr)
    return _tc_update(x, msg[0], msg[1], inv[:_N, None],
                      W1, b1.reshape(1, _D), W2, b2.reshape(1, _D))


# speedup vs baseline: 3.8574x; 1.0235x over previous
import functools

import jax
import jax.numpy as jnp
from jax import lax
from jax.experimental import pallas as pl
from jax.experimental.pallas import tpu as pltpu
from jax.experimental.pallas import tpu_sc as plsc

_N = 10000
_NP = 10240
_E = 160000
_D = 256

_NC = 2
_NS = 16
_DH = _D // 2
_CH = 80
_EPT = _E // _NS
_NCH = _EPT // _CH
_RPT = _NP // _NS
_ZR = 16
_VL = 16


@functools.partial(
    pl.kernel,
    out_type=(
        jax.ShapeDtypeStruct((_NC, _NP, _DH), jnp.float32),
        jax.ShapeDtypeStruct((_NP,), jnp.float32),
    ),
    mesh=plsc.VectorSubcoreMesh(core_axis_name="c", subcore_axis_name="s"),
    scratch_types=[
        pltpu.VMEM((_CH,), jnp.int32),
        pltpu.VMEM((_CH,), jnp.int32),
        pltpu.VMEM((_CH,), jnp.int32),
        pltpu.VMEM((_CH,), jnp.int32),
        pltpu.VMEM((_CH, _DH), jnp.float32),
        pltpu.VMEM((_CH, _DH), jnp.float32),
        pltpu.VMEM((_CH, _DH), jnp.float32),
        pltpu.VMEM((_CH, _DH), jnp.float32),
        pltpu.VMEM((_CH,), jnp.float32),
        pltpu.VMEM((_ZR, _DH), jnp.float32),
        pltpu.VMEM((_RPT,), jnp.float32),
        pltpu.VMEM_SHARED((_NP, _DH), jnp.float32),
        pltpu.VMEM_SHARED((_NP,), jnp.float32),
        pltpu.SemaphoreType.DMA,
        pltpu.SemaphoreType.DMA,
        pltpu.SemaphoreType.DMA,
        pltpu.SemaphoreType.DMA,
        pltpu.SemaphoreType.DMA,
        pltpu.SemaphoreType.DMA,
        pltpu.SemaphoreType.DMA,
        pltpu.SemaphoreType.DMA,
    ],
)
def _sc_aggregate(xs_hbm, src_hbm, dst_hbm, ea_hbm,
                  msg_hbm, inv_hbm,
                  src_v0, dst_v0, src_v1, dst_v1,
                  xj_v0, ea_v0, xj_v1, ea_v1,
                  ones_v, zb_v, cnt_v,
                  acc_s, cnt_s,
                  sem_i0, sem_i1, sem_g0, sem_g1, sem_e0, sem_e1,
                  sem_s, sem_z):
    c = lax.axis_index("c")
    s = lax.axis_index("s")

    zero16 = jnp.zeros((_VL,), jnp.float32)
    one16 = jnp.full((_VL,), 1.0, jnp.float32)

    def fill_zb(i, _):
        r = i // (_DH // _VL)
        o = (i % (_DH // _VL)) * _VL
        zb_v[r, pl.ds(o, _VL)] = zero16
        return 0
    lax.fori_loop(0, _ZR * (_DH // _VL), fill_zb, 0)

    def fill_ones(i, _):
        ones_v[pl.ds(i * _VL, _VL)] = one16
        return 0
    lax.fori_loop(0, _CH // _VL, fill_ones, 0)

    row0 = s * _RPT
    zcps = []
    for q in range(_RPT // _ZR):
        zcps.append(pltpu.async_copy(
            zb_v, acc_s.at[pl.ds(row0 + q * _ZR, _ZR), :], sem_z))
    for q in range(_RPT // _DH):
        zcps.append(pltpu.async_copy(
            zb_v.at[0], cnt_s.at[pl.ds(row0 + q * _DH, _DH)], sem_z))
    for cp in zcps:
        cp.wait()

    plsc.subcore_barrier()

    ebase = s * _EPT
    bufs = (
        (src_v0, dst_v0, xj_v0, ea_v0, sem_i0, sem_g0, sem_e0),
        (src_v1, dst_v1, xj_v1, ea_v1, sem_i1, sem_g1, sem_e1),
    )

    def issue_idx(k, b):
        eb = ebase + k * _CH
        pltpu.async_copy(src_hbm.at[pl.ds(eb, _CH)], b[0], b[4])
        pltpu.async_copy(dst_hbm.at[pl.ds(eb, _CH)], b[1], b[4])

    def wait_idx(b):
        pltpu.make_async_copy(src_hbm.at[pl.ds(0, _CH)], b[0], b[4]).wait()
        pltpu.make_async_copy(dst_hbm.at[pl.ds(0, _CH)], b[1], b[4]).wait()

    def issue_ge(k, b):
        eb = ebase + k * _CH
        pltpu.async_copy(xs_hbm.at[c].at[b[0]], b[2], b[5])
        pltpu.async_copy(ea_hbm.at[pl.ds(eb, _CH), pl.ds(c * _DH, _DH)],
                         b[3], b[6])

    def wait_ge(b):
        pltpu.make_async_copy(xs_hbm.at[c].at[b[0]], b[2], b[5]).wait()
        pltpu.make_async_copy(ea_hbm.at[pl.ds(0, _CH), pl.ds(0, _DH)],
                              b[3], b[6]).wait()

    def chunk_step(k, p):
        b = bufs[p]
        bn = bufs[1 - p]

        @pl.when(k + 1 < _NCH)
        def _():
            wait_idx(bn)
            issue_ge(k + 1, bn)

        wait_ge(b)

        xj_v, ea_v = b[2], b[3]

        def crow(r, _):
            for j in range(_DH // _VL):
                o = j * _VL
                v = xj_v[r, pl.ds(o, _VL)] + ea_v[r, pl.ds(o, _VL)]
                xj_v[r, pl.ds(o, _VL)] = v / (1.0 + jnp.exp(-v))
            return 0
        lax.fori_loop(0, _CH, crow, 0)

        s1 = pltpu.async_copy(xj_v, acc_s.at[b[1]], sem_s, add=True)
        s2 = pltpu.async_copy(ones_v, cnt_s.at[b[1]], sem_s, add=True)
        s1.wait()
        s2.wait()

        @pl.when(k + 2 < _NCH)
        def _():
            issue_idx(k + 2, b)

    pltpu.sync_copy(src_hbm.at[pl.ds(ebase, _CH)], src_v0)
    pltpu.sync_copy(dst_hbm.at[pl.ds(ebase, _CH)], dst_v0)
    issue_ge(0, bufs[0])
    issue_idx(1, bufs[1])

    def pair(j, _):
        chunk_step(2 * j, 0)
        chunk_step(2 * j + 1, 1)
        return 0
    lax.fori_loop(0, (_NCH - 1) // 2, pair, 0)
    chunk_step(_NCH - 1, 0)

    plsc.subcore_barrier()

    pltpu.sync_copy(acc_s.at[pl.ds(row0, _RPT), :],
                    msg_hbm.at[c, pl.ds(row0, _RPT), :])

    @pl.when(c == 0)
    def _():
        pltpu.sync_copy(cnt_s.at[pl.ds(row0, _RPT)], cnt_v)

        def recip(i, _):
            o = i * _VL
            v = cnt_v[pl.ds(o, _VL)]
            cnt_v[pl.ds(o, _VL)] = 1.0 / jnp.maximum(v, 1.0)
            return 0
        lax.fori_loop(0, _RPT // _VL, recip, 0)

        pltpu.sync_copy(cnt_v, inv_hbm.at[pl.ds(row0, _RPT)])


def _tc_body(x_ref, m0_ref, m1_ref, inv_ref, w1_ref, b1_ref, w2_ref, b2_ref,
             o_ref):
    x = x_ref[...]
    aggr = jnp.concatenate([m0_ref[...], m1_ref[...]], axis=1) * inv_ref[...]
    z = x + aggr
    h = jnp.dot(z, w1_ref[...], preferred_element_type=jnp.float32) + b1_ref[...]
    h = h / (1.0 + jnp.exp(-h))
    h = jnp.dot(h, w2_ref[...], preferred_element_type=jnp.float32) + b2_ref[...]
    o_ref[...] = x + h


_TB = 1000


def _tc_update(x, m0, m1, inv, W1, b1, W2, b2):
    grid = (_N // _TB,)
    return pl.pallas_call(
        _tc_body,
        grid=grid,
        in_specs=[
            pl.BlockSpec((_TB, _D), lambda i: (i, 0)),
            pl.BlockSpec((_TB, _DH), lambda i: (i, 0)),
            pl.BlockSpec((_TB, _DH), lambda i: (i, 0)),
            pl.BlockSpec((_TB, 1), lambda i: (i, 0)),
            pl.BlockSpec((_D, _D), lambda i: (0, 0)),
            pl.BlockSpec((1, _D), lambda i: (0, 0)),
            pl.BlockSpec((_D, _D), lambda i: (0, 0)),
            pl.BlockSpec((1, _D), lambda i: (0, 0)),
        ],
        out_specs=pl.BlockSpec((_TB, _D), lambda i: (i, 0)),
        out_shape=jax.ShapeDtypeStruct((_N, _D), jnp.float32),
    )(x, m0, m1, inv, W1, b1, W2, b2)


def kernel(x, edge_index, edge_attr, ln_scale, ln_bias, W1, b1, W2, b2):
    xs = jnp.transpose(x.reshape(_N, _NC, _DH), (1, 0, 2))
    src = edge_index[0]
    dst = edge_index[1]
    msg, inv = _sc_aggregate(xs, src, dst, edge_attr)
    return _tc_update(x, msg[0], msg[1], inv[:_N, None],
                      W1, b1.reshape(1, _D), W2, b2.reshape(1, _D))


# speedup vs baseline: 4.0293x; 1.0446x over previous
import functools

import jax
import jax.numpy as jnp
from jax import lax
from jax.experimental import pallas as pl
from jax.experimental.pallas import tpu as pltpu
from jax.experimental.pallas import tpu_sc as plsc

_N = 10000
_NP = 10240
_E = 160000
_D = 256

_NC = 2
_NS = 16
_DH = _D // 2
_CH = 80
_EPT = _E // _NS
_NCH = _EPT // _CH
_RPT = _NP // _NS
_ZR = 16
_VL = 16


@functools.partial(
    pl.kernel,
    out_type=(
        jax.ShapeDtypeStruct((_NC, _NP, _DH), jnp.float32),
        jax.ShapeDtypeStruct((_NP,), jnp.float32),
    ),
    mesh=plsc.VectorSubcoreMesh(core_axis_name="c", subcore_axis_name="s"),
    scratch_types=[
        pltpu.VMEM((_CH,), jnp.int32),
        pltpu.VMEM((_CH // 2,), jnp.int32),
        pltpu.VMEM((_CH // 2,), jnp.int32),
        pltpu.VMEM((_CH,), jnp.int32),
        pltpu.VMEM((_CH // 2,), jnp.int32),
        pltpu.VMEM((_CH // 2,), jnp.int32),
        pltpu.VMEM((_CH, _DH), jnp.float32),
        pltpu.VMEM((_CH, _DH), jnp.float32),
        pltpu.VMEM((_CH, _DH), jnp.float32),
        pltpu.VMEM((_CH, _DH), jnp.float32),
        pltpu.VMEM((_CH // 2,), jnp.float32),
        pltpu.VMEM((_ZR, _DH), jnp.float32),
        pltpu.VMEM((_RPT,), jnp.float32),
        pltpu.VMEM_SHARED((_NP, _DH), jnp.float32),
        pltpu.VMEM_SHARED((_NP,), jnp.float32),
        pltpu.SemaphoreType.DMA,
        pltpu.SemaphoreType.DMA,
        pltpu.SemaphoreType.DMA,
        pltpu.SemaphoreType.DMA,
        pltpu.SemaphoreType.DMA,
        pltpu.SemaphoreType.DMA,
        pltpu.SemaphoreType.DMA,
        pltpu.SemaphoreType.DMA,
    ],
)
def _sc_aggregate(xs_hbm, src_hbm, dst_hbm, ea_hbm,
                  msg_hbm, inv_hbm,
                  src_v0, dlo_v0, dhi_v0, src_v1, dlo_v1, dhi_v1,
                  xj_v0, ea_v0, xj_v1, ea_v1,
                  ones_v, zb_v, cnt_v,
                  acc_s, cnt_s,
                  sem_i0, sem_i1, sem_g0, sem_g1, sem_e0, sem_e1,
                  sem_s, sem_z):
    c = lax.axis_index("c")
    s = lax.axis_index("s")

    zero16 = jnp.zeros((_VL,), jnp.float32)
    one16 = jnp.full((_VL,), 1.0, jnp.float32)

    def fill_zb(i, _):
        r = i // (_DH // _VL)
        o = (i % (_DH // _VL)) * _VL
        zb_v[r, pl.ds(o, _VL)] = zero16
        return 0
    lax.fori_loop(0, _ZR * (_DH // _VL), fill_zb, 0)

    ones_v[pl.ds(0, _VL)] = one16
    ones_v[pl.ds(_VL, _VL)] = one16
    ones_v[pl.ds(_CH // 2 - _VL, _VL)] = one16

    row0 = s * _RPT
    zcps = []
    for q in range(_RPT // _ZR):
        zcps.append(pltpu.async_copy(
            zb_v, acc_s.at[pl.ds(row0 + q * _ZR, _ZR), :], sem_z))
    for q in range(_RPT // _DH):
        zcps.append(pltpu.async_copy(
            zb_v.at[0], cnt_s.at[pl.ds(row0 + q * _DH, _DH)], sem_z))
    for cp in zcps:
        cp.wait()

    plsc.subcore_barrier()

    ebase = s * _EPT
    bufs = (
        (src_v0, dlo_v0, xj_v0, ea_v0, sem_i0, sem_g0, sem_e0, dhi_v0),
        (src_v1, dlo_v1, xj_v1, ea_v1, sem_i1, sem_g1, sem_e1, dhi_v1),
    )
    _HC = _CH // 2

    def issue_idx(k, b):
        eb = ebase + k * _CH
        pltpu.async_copy(src_hbm.at[pl.ds(eb, _CH)], b[0], b[4])
        pltpu.async_copy(dst_hbm.at[pl.ds(eb, _HC)], b[1], b[4])
        pltpu.async_copy(dst_hbm.at[pl.ds(eb + _HC, _HC)], b[7], b[4])

    def wait_idx(b):
        pltpu.make_async_copy(src_hbm.at[pl.ds(0, _CH)], b[0], b[4]).wait()
        pltpu.make_async_copy(dst_hbm.at[pl.ds(0, _HC)], b[1], b[4]).wait()
        pltpu.make_async_copy(dst_hbm.at[pl.ds(0, _HC)], b[7], b[4]).wait()

    def issue_ge(k, b):
        eb = ebase + k * _CH
        pltpu.async_copy(xs_hbm.at[c].at[b[0]], b[2], b[5])
        pltpu.async_copy(ea_hbm.at[pl.ds(eb, _CH), pl.ds(c * _DH, _DH)],
                         b[3], b[6])

    def wait_ge(b):
        pltpu.make_async_copy(xs_hbm.at[c].at[b[0]], b[2], b[5]).wait()
        pltpu.make_async_copy(ea_hbm.at[pl.ds(0, _CH), pl.ds(0, _DH)],
                              b[3], b[6]).wait()

    def chunk_step(k, p):
        b = bufs[p]
        bn = bufs[1 - p]

        @pl.when(k + 1 < _NCH)
        def _():
            wait_idx(bn)
            issue_ge(k + 1, bn)

        wait_ge(b)

        xj_v, ea_v = b[2], b[3]

        def crow(r, _):
            for j in range(_DH // _VL):
                o = j * _VL
                v = xj_v[r, pl.ds(o, _VL)] + ea_v[r, pl.ds(o, _VL)]
                xj_v[r, pl.ds(o, _VL)] = v / (1.0 + jnp.exp(-v))
            return 0
        lax.fori_loop(0, _HC, crow, 0)

        s1 = pltpu.async_copy(xj_v.at[pl.ds(0, _HC), :], acc_s.at[b[1]],
                              sem_s, add=True)
        s2 = pltpu.async_copy(ones_v, cnt_s.at[b[1]], sem_s, add=True)

        lax.fori_loop(_HC, _CH, crow, 0)

        s1.wait()
        s2.wait()
        s3 = pltpu.async_copy(xj_v.at[pl.ds(_HC, _HC), :], acc_s.at[b[7]],
                              sem_s, add=True)
        s4 = pltpu.async_copy(ones_v, cnt_s.at[b[7]], sem_s, add=True)
        s3.wait()
        s4.wait()

        @pl.when(k + 2 < _NCH)
        def _():
            issue_idx(k + 2, b)

    pltpu.sync_copy(src_hbm.at[pl.ds(ebase, _CH)], src_v0)
    pltpu.sync_copy(dst_hbm.at[pl.ds(ebase, _HC)], dlo_v0)
    pltpu.sync_copy(dst_hbm.at[pl.ds(ebase + _HC, _HC)], dhi_v0)
    issue_ge(0, bufs[0])
    issue_idx(1, bufs[1])

    def pair(j, _):
        chunk_step(2 * j, 0)
        chunk_step(2 * j + 1, 1)
        return 0
    lax.fori_loop(0, (_NCH - 1) // 2, pair, 0)
    chunk_step(_NCH - 1, 0)

    plsc.subcore_barrier()

    pltpu.sync_copy(acc_s.at[pl.ds(row0, _RPT), :],
                    msg_hbm.at[c, pl.ds(row0, _RPT), :])

    @pl.when(c == 0)
    def _():
        pltpu.sync_copy(cnt_s.at[pl.ds(row0, _RPT)], cnt_v)

        def recip(i, _):
            o = i * _VL
            v = cnt_v[pl.ds(o, _VL)]
            cnt_v[pl.ds(o, _VL)] = 1.0 / jnp.maximum(v, 1.0)
            return 0
        lax.fori_loop(0, _RPT // _VL, recip, 0)

        pltpu.sync_copy(cnt_v, inv_hbm.at[pl.ds(row0, _RPT)])


def _tc_body(x_ref, m0_ref, m1_ref, inv_ref, w1_ref, b1_ref, w2_ref, b2_ref,
             o_ref):
    x = x_ref[...]
    aggr = jnp.concatenate([m0_ref[...], m1_ref[...]], axis=1) * inv_ref[...]
    z = x + aggr
    h = jnp.dot(z, w1_ref[...], preferred_element_type=jnp.float32) + b1_ref[...]
    h = h / (1.0 + jnp.exp(-h))
    h = jnp.dot(h, w2_ref[...], preferred_element_type=jnp.float32) + b2_ref[...]
    o_ref[...] = x + h


_TB = 1000


def _tc_update(x, m0, m1, inv, W1, b1, W2, b2):
    grid = (_N // _TB,)
    return pl.pallas_call(
        _tc_body,
        grid=grid,
        in_specs=[
            pl.BlockSpec((_TB, _D), lambda i: (i, 0)),
            pl.BlockSpec((_TB, _DH), lambda i: (i, 0)),
            pl.BlockSpec((_TB, _DH), lambda i: (i, 0)),
            pl.BlockSpec((_TB, 1), lambda i: (i, 0)),
            pl.BlockSpec((_D, _D), lambda i: (0, 0)),
            pl.BlockSpec((1, _D), lambda i: (0, 0)),
            pl.BlockSpec((_D, _D), lambda i: (0, 0)),
            pl.BlockSpec((1, _D), lambda i: (0, 0)),
        ],
        out_specs=pl.BlockSpec((_TB, _D), lambda i: (i, 0)),
        out_shape=jax.ShapeDtypeStruct((_N, _D), jnp.float32),
    )(x, m0, m1, inv, W1, b1, W2, b2)


def kernel(x, edge_index, edge_attr, ln_scale, ln_bias, W1, b1, W2, b2):
    xs = jnp.transpose(x.reshape(_N, _NC, _DH), (1, 0, 2))
    src = edge_index[0]
    dst = edge_index[1]
    msg, inv = _sc_aggregate(xs, src, dst, edge_attr)
    return _tc_update(x, msg[0], msg[1], inv[:_N, None],
                      W1, b1.reshape(1, _D), W2, b2.reshape(1, _D))
